# baseline (speedup 1.0000x reference)
import functools

import jax
import jax.numpy as jnp
from jax.experimental import pallas as pl
from jax.experimental.pallas import tpu as pltpu


def _pick_block(n: int, cap: int) -> int:
    best = 0
    for d in range(8, min(n, cap) + 1, 8):
        if n % d == 0:
            best = d
    return best if best > 0 else n


def _v_body(x_ref, w1_ref, v_ref, *, C: int, H: int):
    for k in range(C):
        v = jnp.dot(x_ref[k], w1_ref[...], preferred_element_type=jnp.float32)
        v_ref[:, k * H:(k + 1) * H] = v.astype(jnp.bfloat16)


def _uni_body(hg_ref, v_ref, vi_ref, b1_ref, w2_ref, u_ref, hgb_ref):
    hgb = hg_ref[...].astype(jnp.bfloat16)
    hgb_ref[...] = hgb
    acc = jnp.dot(hgb, v_ref[...], preferred_element_type=jnp.float32)
    h = acc + vi_ref[...].astype(jnp.float32) + b1_ref[...]
    h = jnp.maximum(h, 0.0)
    u_ref[...] = jnp.dot(h, w2_ref[...], preferred_element_type=jnp.float32)


def _out_body(hg_ref, u_ref, ui_ref, b2_ref, o_ref):
    acc = jnp.dot(hg_ref[...], u_ref[...].astype(jnp.bfloat16),
                  preferred_element_type=jnp.float32)
    o_ref[...] = acc + ui_ref[...] + b2_ref[...]


def kernel(x_list, hg, W1, b1, W2, b2):
    C, N, F = x_list.shape
    H = W1.shape[1]
    CH = C * H
    O = W2.shape[1]

    bma = _pick_block(N, 2048)
    bm = _pick_block(N, 256)

    b1c = jnp.tile(b1, C).reshape(1, CH)
    b2r = b2.reshape(1, O)

    V = pl.pallas_call(
        functools.partial(_v_body, C=C, H=H),
        grid=(N // bma,),
        in_specs=[
            pl.BlockSpec((C, bma, F), lambda i: (0, i, 0)),
            pl.BlockSpec((F, H), lambda i: (0, 0)),
        ],
        out_specs=pl.BlockSpec((bma, CH), lambda i: (i, 0)),
        out_shape=jax.ShapeDtypeStruct((N, CH), jnp.bfloat16),
        compiler_params=pltpu.CompilerParams(
            dimension_semantics=("parallel",)),
    )(x_list, W1)

    u, hgb = pl.pallas_call(
        _uni_body,
        grid=(N // bm,),
        in_specs=[
            pl.BlockSpec((bm, N), lambda i: (i, 0)),
            pl.BlockSpec((N, CH), lambda i: (0, 0)),
            pl.BlockSpec((bm, CH), lambda i: (i, 0)),
            pl.BlockSpec((1, CH), lambda i: (0, 0)),
            pl.BlockSpec((CH, O), lambda i: (0, 0)),
        ],
        out_specs=[
            pl.BlockSpec((bm, O), lambda i: (i, 0)),
            pl.BlockSpec((bm, N), lambda i: (i, 0)),
        ],
        out_shape=[
            jax.ShapeDtypeStruct((N, O), jnp.float32),
            jax.ShapeDtypeStruct((N, N), jnp.bfloat16),
        ],
        compiler_params=pltpu.CompilerParams(
            dimension_semantics=("parallel",)),
    )(hg, V, V, b1c, W2)

    out = pl.pallas_call(
        _out_body,
        grid=(N // bm,),
        in_specs=[
            pl.BlockSpec((bm, N), lambda i: (i, 0)),
            pl.BlockSpec((N, O), lambda i: (0, 0)),
            pl.BlockSpec((bm, O), lambda i: (i, 0)),
            pl.BlockSpec((1, O), lambda i: (0, 0)),
        ],
        out_specs=pl.BlockSpec((bm, O), lambda i: (i, 0)),
        out_shape=jax.ShapeDtypeStruct((N, O), jnp.float32),
        compiler_params=pltpu.CompilerParams(
            dimension_semantics=("parallel",)),
    )(hgb, u, u, b2r)

    return out


# speedup vs baseline: 1.1296x; 1.1296x over previous
import functools

import jax
import jax.numpy as jnp
from jax.experimental import pallas as pl
from jax.experimental.pallas import tpu as pltpu

_S = 2


def _pick_block(n: int, cap: int) -> int:
    best = 0
    for d in range(8, min(n, cap) + 1, 8):
        if n % d == 0:
            best = d
    return best if best > 0 else n


def _v_body(x_ref, w1_ref, v_ref, *, C: int, H: int):
    for k in range(C):
        v = jnp.dot(x_ref[k], w1_ref[...], preferred_element_type=jnp.float32)
        v_ref[:, k * H:(k + 1) * H] = v.astype(jnp.bfloat16)


def _uni_body(*refs, bm: int):
    hg_refs = refs[:_S]
    v_ref = refs[_S]
    vi_ref = refs[_S + 1]
    b1_ref = refs[_S + 2]
    w2_ref = refs[_S + 3]
    u_ref = refs[_S + 4]
    for j in range(_S):
        acc = jnp.dot(hg_refs[j][...].astype(jnp.bfloat16), v_ref[...],
                      preferred_element_type=jnp.float32)
        h = acc + vi_ref[j * bm:(j + 1) * bm, :].astype(jnp.float32) \
            + b1_ref[...]
        h = jnp.maximum(h, 0.0)
        u_ref[j * bm:(j + 1) * bm, :] = jnp.dot(
            h, w2_ref[...], preferred_element_type=jnp.float32)


def _out_body(*refs, bm: int):
    hg_refs = refs[:_S]
    u_ref = refs[_S]
    ui_ref = refs[_S + 1]
    b2_ref = refs[_S + 2]
    o_ref = refs[_S + 3]
    ub = u_ref[...].astype(jnp.bfloat16)
    for j in range(_S):
        acc = jnp.dot(hg_refs[j][...].astype(jnp.bfloat16), ub,
                      preferred_element_type=jnp.float32)
        o_ref[j * bm:(j + 1) * bm, :] = (
            acc + ui_ref[j * bm:(j + 1) * bm, :] + b2_ref[...])


def kernel(x_list, hg, W1, b1, W2, b2):
    C, N, F = x_list.shape
    H = W1.shape[1]
    CH = C * H
    O = W2.shape[1]

    bma = _pick_block(N, 2048)
    bm = _pick_block(N // _S, 256)
    steps = N // (_S * bm)

    b1c = jnp.tile(b1, C).reshape(1, CH)
    b2r = b2.reshape(1, O)

    V = pl.pallas_call(
        functools.partial(_v_body, C=C, H=H),
        grid=(N // bma,),
        in_specs=[
            pl.BlockSpec((C, bma, F), lambda i: (0, i, 0)),
            pl.BlockSpec((F, H), lambda i: (0, 0)),
        ],
        out_specs=pl.BlockSpec((bma, CH), lambda i: (i, 0)),
        out_shape=jax.ShapeDtypeStruct((N, CH), jnp.bfloat16),
        compiler_params=pltpu.CompilerParams(
            dimension_semantics=("parallel",)),
    )(x_list, W1)

    def _stripe_spec(j):
        return pl.BlockSpec((bm, N), lambda i, j=j: (_S * i + j, 0))

    u = pl.pallas_call(
        functools.partial(_uni_body, bm=bm),
        grid=(steps,),
        in_specs=[_stripe_spec(j) for j in range(_S)] + [
            pl.BlockSpec((N, CH), lambda i: (0, 0)),
            pl.BlockSpec((_S * bm, CH), lambda i: (i, 0)),
            pl.BlockSpec((1, CH), lambda i: (0, 0)),
            pl.BlockSpec((CH, O), lambda i: (0, 0)),
        ],
        out_specs=pl.BlockSpec((_S * bm, O), lambda i: (i, 0)),
        out_shape=jax.ShapeDtypeStruct((N, O), jnp.float32),
        compiler_params=pltpu.CompilerParams(
            dimension_semantics=("parallel",)),
    )(*([hg] * _S), V, V, b1c, W2)

    out = pl.pallas_call(
        functools.partial(_out_body, bm=bm),
        grid=(steps,),
        in_specs=[_stripe_spec(j) for j in range(_S)] + [
            pl.BlockSpec((N, O), lambda i: (0, 0)),
            pl.BlockSpec((_S * bm, O), lambda i: (i, 0)),
            pl.BlockSpec((1, O), lambda i: (0, 0)),
        ],
        out_specs=pl.BlockSpec((_S * bm, O), lambda i: (i, 0)),
        out_shape=jax.ShapeDtypeStruct((N, O), jnp.float32),
        compiler_params=pltpu.CompilerParams(
            dimension_semantics=("parallel",)),
    )(*([hg] * _S), u, u, b2r)

    return out
